# baseline (speedup 1.0000x reference)
import functools

import jax
import jax.numpy as jnp
from jax import lax
from jax.experimental import pallas as pl
from jax.experimental.pallas import tpu as pltpu
from jax.experimental.pallas import tpu_sc as plsc

_VOCAB = 1000000
_D = 64
_P = 128
_B = 1024
_DOC_LEN = 200
_QUERY_LEN = 50

_NC = 2
_NS = 16
_NW = _NC * _NS
_BPW = _B // _NW
_DCH = 100
_DCHUNKS = _DOC_LEN // _DCH
_NDC = _BPW * _DCHUNKS
_NBUF = 4


_VB = 128
_NVB = _VOCAB // _VB
_VTAIL = _VOCAB - _NVB * _VB


def _transpose_block(in_ref, out_ref, width):
    iota = lax.iota(jnp.int32, 16)
    for c in range(_D):
        cvec = jnp.full((16,), c, dtype=jnp.int32)
        for v0 in range(0, width, 16):
            plsc.store_scatter(out_ref, [iota + v0, cvec],
                               in_ref[c, pl.ds(v0, 16)])


def _sc_relayout_kernel(tt_hbm, aux_hbm, out_hbm, in0, in1, out0, out1,
                        aux_v, isem0, isem1, osem0, osem1):
    wid = lax.axis_index("s") * _NC + lax.axis_index("c")
    ins = (in0, in1)
    outs = (out0, out1)
    isems = (isem0, isem1)
    osems = (osem0, osem1)
    nrounds = _NVB // _NW + 1

    def i_start(j, p):
        off = pl.multiple_of(_VB * j, _VB)
        return pltpu.async_copy(tt_hbm.at[:, pl.ds(off, _VB)], ins[p],
                                isems[p])

    def i_wait(j, p):
        off = pl.multiple_of(_VB * j, _VB)
        pltpu.make_async_copy(tt_hbm.at[:, pl.ds(off, _VB)], ins[p],
                              isems[p]).wait()

    def o_start(j, p):
        off = pl.multiple_of(_VB * j, _VB)
        return pltpu.async_copy(
            outs[p], out_hbm.at[pl.ds(off, _VB)], osems[p])

    def o_wait(j, p):
        off = pl.multiple_of(_VB * j, _VB)
        pltpu.make_async_copy(
            outs[p], out_hbm.at[pl.ds(off, _VB)], osems[p]).wait()

    i_start(wid, 0)

    def rnd2(k2, carry):
        for p in range(2):
            k = 2 * k2 + p
            j = wid + _NW * k

            @pl.when(j < _NVB)
            def _():
                i_wait(j, p)

                @pl.when(j + _NW < _NVB)
                def _():
                    i_start(j + _NW, 1 - p)

                @pl.when(k >= 2)
                def _():
                    o_wait(j - 2 * _NW, p)

                _transpose_block(ins[p], outs[p], _VB)
                o_start(j, p)

        return carry

    lax.fori_loop(0, (nrounds + 1) // 2, rnd2, 0, unroll=False)

    last_k = lax.div(_NVB - 1 - wid, _NW)
    for p in range(2):
        kp = last_k - lax.rem(last_k - p + 2, 2)

        @pl.when(kp >= 0)
        def _():
            o_wait(wid + _NW * kp, p)

    @pl.when(wid == 0)
    def _():
        pltpu.sync_copy(aux_hbm, aux_v)
        for r in range(_VTAIL):
            for g in range(4):
                out0[r, pl.ds(16 * g, 16)] = aux_v[r, pl.ds(16 * g, 16)]
        pltpu.sync_copy(out0.at[pl.ds(0, _VTAIL)],
                        out_hbm.at[pl.ds(_VOCAB - _VTAIL, _VTAIL)])


def _sc_relayout(table_t, aux):
    mesh = plsc.VectorSubcoreMesh(core_axis_name="c", subcore_axis_name="s")
    fn = functools.partial(
        pl.kernel,
        mesh=mesh,
        compiler_params=pltpu.CompilerParams(needs_layout_passes=False),
        out_type=jax.ShapeDtypeStruct((_VOCAB, _P), jnp.float32),
        scratch_types=[
            pltpu.VMEM((_D, _VB), jnp.float32),
            pltpu.VMEM((_D, _VB), jnp.float32),
            pltpu.VMEM((_VB, _P), jnp.float32),
            pltpu.VMEM((_VB, _P), jnp.float32),
            pltpu.VMEM((_VTAIL, _D), jnp.float32),
            pltpu.SemaphoreType.DMA,
            pltpu.SemaphoreType.DMA,
            pltpu.SemaphoreType.DMA,
            pltpu.SemaphoreType.DMA,
        ],
    )(_sc_relayout_kernel)
    return fn(table_t, aux)


def _pool_chunk(rows_ref, n_rows, acc):
    def add_row(a, r):
        a0, a1, a2, a3 = a
        a0 = a0 + rows_ref[r, pl.ds(0, 16)]
        a1 = a1 + rows_ref[r, pl.ds(16, 16)]
        a2 = a2 + rows_ref[r, pl.ds(32, 16)]
        a3 = a3 + rows_ref[r, pl.ds(48, 16)]
        return (a0, a1, a2, a3)

    def body(j, a):
        r0 = 4 * j
        for k in range(4):
            a = add_row(a, r0 + k)
        return a

    acc = lax.fori_loop(0, n_rows // 4, body, acc, unroll=False)
    for r in range(n_rows - n_rows % 4, n_rows):
        acc = add_row(acc, r)
    return acc


def _store_acc(acc_ref, i, acc):
    a0, a1, a2, a3 = acc
    acc_ref[i, pl.ds(0, 16)] = a0
    acc_ref[i, pl.ds(16, 16)] = a1
    acc_ref[i, pl.ds(32, 16)] = a2
    acc_ref[i, pl.ds(48, 16)] = a3


def _sc_pool_kernel(didx_hbm, qidx_hbm, table_hbm, d_out_hbm, q_out_hbm,
                    didx_v, qidx_v, rows0, rows1, rows2, rows3,
                    dacc_v, qacc_v, sem0, sem1, sem2, sem3):
    wid = lax.axis_index("s") * _NC + lax.axis_index("c")
    rows = (rows0, rows1, rows2, rows3)
    sems = (sem0, sem1, sem2, sem3)

    pltpu.sync_copy(didx_hbm.at[pl.ds(wid * _NDC, _NDC)], didx_v)
    pltpu.sync_copy(qidx_hbm.at[pl.ds(wid * _BPW, _BPW)], qidx_v)

    zero = jnp.zeros((16,), jnp.float32)
    z4 = (zero, zero, zero, zero)

    def d_start(chunk, b):
        return pltpu.async_copy(table_hbm.at[didx_v.at[chunk]], rows[b],
                                sems[b])

    def d_wait(chunk, b):
        pltpu.make_async_copy(table_hbm.at[didx_v.at[chunk]], rows[b],
                              sems[b]).wait()

    for b in range(_NBUF):
        d_start(b, b)

    def d_round(k, carry):
        acc = z4
        for b in range(_NBUF):
            chunk = _NBUF * k + b
            d_wait(chunk, b)
            acc = _pool_chunk(rows[b], _DCH, acc)
            if b % _DCHUNKS == _DCHUNKS - 1:
                _store_acc(dacc_v, 2 * k + b // _DCHUNKS, acc)
                acc = z4

            @pl.when(k < _NDC // _NBUF - 1)
            def _():
                d_start(chunk + _NBUF, b)

        return carry

    lax.fori_loop(0, _NDC // _NBUF, d_round, 0, unroll=False)
    pltpu.sync_copy(dacc_v, d_out_hbm.at[pl.ds(wid * _BPW, _BPW)])

    def q_start(i, b):
        return pltpu.async_copy(table_hbm.at[qidx_v.at[i]],
                                rows[b].at[pl.ds(0, _QUERY_LEN)], sems[b])

    def q_wait(i, b):
        pltpu.make_async_copy(table_hbm.at[qidx_v.at[i]],
                              rows[b].at[pl.ds(0, _QUERY_LEN)],
                              sems[b]).wait()

    for b in range(_NBUF):
        q_start(b, b)

    def q_round(k, carry):
        for b in range(_NBUF):
            i = _NBUF * k + b
            q_wait(i, b)
            acc = _pool_chunk(rows[b], _QUERY_LEN, z4)
            _store_acc(qacc_v, i, acc)

            @pl.when(k < _BPW // _NBUF - 1)
            def _():
                q_start(i + _NBUF, b)

        return carry

    lax.fori_loop(0, _BPW // _NBUF, q_round, 0, unroll=False)
    pltpu.sync_copy(qacc_v, q_out_hbm.at[pl.ds(wid * _BPW, _BPW)])


def _sc_pool(didx, qidx, table2):
    mesh = plsc.VectorSubcoreMesh(core_axis_name="c", subcore_axis_name="s")
    fn = functools.partial(
        pl.kernel,
        mesh=mesh,
        out_type=[
            jax.ShapeDtypeStruct((_B, _D), jnp.float32),
            jax.ShapeDtypeStruct((_B, _D), jnp.float32),
        ],
        scratch_types=[
            pltpu.VMEM((_NDC, _DCH), jnp.int32),
            pltpu.VMEM((_BPW, _QUERY_LEN), jnp.int32),
            pltpu.VMEM((_DCH, _P), jnp.float32),
            pltpu.VMEM((_DCH, _P), jnp.float32),
            pltpu.VMEM((_DCH, _P), jnp.float32),
            pltpu.VMEM((_DCH, _P), jnp.float32),
            pltpu.VMEM((_BPW, _D), jnp.float32),
            pltpu.VMEM((_BPW, _D), jnp.float32),
            pltpu.SemaphoreType.DMA,
            pltpu.SemaphoreType.DMA,
            pltpu.SemaphoreType.DMA,
            pltpu.SemaphoreType.DMA,
        ],
    )(_sc_pool_kernel)
    return fn(didx, qidx, table2)


def _tc_head_kernel(d_ref, q_ref, dw1_ref, db1_ref, dw2_ref, db2_ref,
                    qw1_ref, qb1_ref, qw2_ref, qb2_ref, out_ref):
    def dot_t(a, w):
        return lax.dot_general(a, w, (((1,), (1,)), ((), ())),
                               preferred_element_type=jnp.float32)

    d = d_ref[...] * (1.0 / _DOC_LEN)
    q = q_ref[...] * (1.0 / _QUERY_LEN)
    dh = jnp.maximum(dot_t(d, dw1_ref[...]) + db1_ref[...], 0.0)
    dp = dot_t(dh, dw2_ref[...]) + db2_ref[...]
    qh = jnp.maximum(dot_t(q, qw1_ref[...]) + qb1_ref[...], 0.0)
    qp = dot_t(qh, qw2_ref[...]) + qb2_ref[...]
    dn = jnp.maximum(jnp.sqrt(jnp.sum(dp * dp, axis=1, keepdims=True)), 1e-8)
    qn = jnp.maximum(jnp.sqrt(jnp.sum(qp * qp, axis=1, keepdims=True)), 1e-8)
    out_ref[...] = jnp.sum(dp * qp, axis=1, keepdims=True) / (dn * qn)


def _tc_head(d_sum, q_sum, d_w1, d_b1, d_w2, d_b2, q_w1, q_b1, q_w2, q_b2):
    return pl.pallas_call(
        _tc_head_kernel,
        out_shape=jax.ShapeDtypeStruct((_B, 1), jnp.float32),
    )(d_sum, q_sum, d_w1, d_b1.reshape(1, _P), d_w2, d_b2.reshape(1, _P),
      q_w1, q_b1.reshape(1, _D), q_w2, q_b2.reshape(1, _P))


def kernel(doc_ids, query_ids, table, d_w1, d_b1, d_w2, d_b2,
           q_w1, q_b1, q_w2, q_b2):
    doc_ids = doc_ids.astype(jnp.int32)
    query_ids = query_ids.astype(jnp.int32)
    table2 = _sc_relayout(table.T, table[_NVB * _VB:])
    didx = doc_ids.reshape(_B * _DCHUNKS, _DCH)
    d_sum, q_sum = _sc_pool(didx, query_ids, table2)
    sim = _tc_head(d_sum, q_sum, d_w1, d_b1, d_w2, d_b2,
                   q_w1, q_b1, q_w2, q_b2)
    return sim.reshape(_B)


# speedup vs baseline: 1.3475x; 1.3475x over previous
import functools

import jax
import jax.numpy as jnp
from jax import lax
from jax.experimental import pallas as pl
from jax.experimental.pallas import tpu as pltpu
from jax.experimental.pallas import tpu_sc as plsc

_VOCAB = 1000000
_D = 64
_P = 128
_B = 1024
_DOC_LEN = 200
_QUERY_LEN = 50

_NC = 2
_NS = 16
_NW = _NC * _NS
_BPW = _B // _NW
_DCH = 100
_DCHUNKS = _DOC_LEN // _DCH
_NDC = _BPW * _DCHUNKS
_NBUF = 4


_VB = 128
_NVB = _VOCAB // _VB
_VTAIL = _VOCAB - _NVB * _VB


def _transpose_block(in_ref, out_ref, width):
    iota = lax.iota(jnp.int32, 16)
    for c0 in range(0, _D, 16):
        rvec = iota + c0
        for v0 in range(0, width, 8):
            vals = [plsc.load_gather(in_ref,
                                     [rvec, jnp.full((16,), v0 + i,
                                                     dtype=jnp.int32)])
                    for i in range(8)]
            for i in range(8):
                out_ref[v0 + i, pl.ds(c0, 16)] = vals[i]


def _sc_relayout_kernel(tt_hbm, aux_hbm, out_hbm, in0, in1, out0, out1,
                        aux_v, isem0, isem1, osem0, osem1):
    wid = lax.axis_index("s") * _NC + lax.axis_index("c")
    ins = (in0, in1)
    outs = (out0, out1)
    isems = (isem0, isem1)
    osems = (osem0, osem1)
    nrounds = _NVB // _NW + 1

    def i_start(j, p):
        off = pl.multiple_of(_VB * j, _VB)
        return pltpu.async_copy(tt_hbm.at[:, pl.ds(off, _VB)], ins[p],
                                isems[p])

    def i_wait(j, p):
        off = pl.multiple_of(_VB * j, _VB)
        pltpu.make_async_copy(tt_hbm.at[:, pl.ds(off, _VB)], ins[p],
                              isems[p]).wait()

    def o_start(j, p):
        off = pl.multiple_of(_VB * j, _VB)
        return pltpu.async_copy(
            outs[p], out_hbm.at[pl.ds(off, _VB)], osems[p])

    def o_wait(j, p):
        off = pl.multiple_of(_VB * j, _VB)
        pltpu.make_async_copy(
            outs[p], out_hbm.at[pl.ds(off, _VB)], osems[p]).wait()

    i_start(wid, 0)

    def rnd2(k2, carry):
        for p in range(2):
            k = 2 * k2 + p
            j = wid + _NW * k

            @pl.when(j < _NVB)
            def _():
                i_wait(j, p)

                @pl.when(j + _NW < _NVB)
                def _():
                    i_start(j + _NW, 1 - p)

                @pl.when(k >= 2)
                def _():
                    o_wait(j - 2 * _NW, p)

                _transpose_block(ins[p], outs[p], _VB)
                o_start(j, p)

        return carry

    lax.fori_loop(0, (nrounds + 1) // 2, rnd2, 0, unroll=False)

    last_k = lax.div(_NVB - 1 - wid, _NW)
    for p in range(2):
        kp = last_k - lax.rem(last_k - p + 2, 2)

        @pl.when(kp >= 0)
        def _():
            o_wait(wid + _NW * kp, p)

    @pl.when(wid == 0)
    def _():
        pltpu.sync_copy(aux_hbm, aux_v)
        for r in range(_VTAIL):
            for g in range(4):
                out0[r, pl.ds(16 * g, 16)] = aux_v[r, pl.ds(16 * g, 16)]
        pltpu.sync_copy(out0.at[pl.ds(0, _VTAIL)],
                        out_hbm.at[pl.ds(_VOCAB - _VTAIL, _VTAIL)])


def _sc_relayout(table_t, aux):
    mesh = plsc.VectorSubcoreMesh(core_axis_name="c", subcore_axis_name="s")
    fn = functools.partial(
        pl.kernel,
        mesh=mesh,
        compiler_params=pltpu.CompilerParams(needs_layout_passes=False),
        out_type=jax.ShapeDtypeStruct((_VOCAB, _P), jnp.float32),
        scratch_types=[
            pltpu.VMEM((_D, _VB), jnp.float32),
            pltpu.VMEM((_D, _VB), jnp.float32),
            pltpu.VMEM((_VB, _P), jnp.float32),
            pltpu.VMEM((_VB, _P), jnp.float32),
            pltpu.VMEM((_VTAIL, _D), jnp.float32),
            pltpu.SemaphoreType.DMA,
            pltpu.SemaphoreType.DMA,
            pltpu.SemaphoreType.DMA,
            pltpu.SemaphoreType.DMA,
        ],
    )(_sc_relayout_kernel)
    return fn(table_t, aux)


def _pool_chunk(rows_ref, n_rows, acc):
    def add_row(a, r):
        a0, a1, a2, a3 = a
        a0 = a0 + rows_ref[r, pl.ds(0, 16)]
        a1 = a1 + rows_ref[r, pl.ds(16, 16)]
        a2 = a2 + rows_ref[r, pl.ds(32, 16)]
        a3 = a3 + rows_ref[r, pl.ds(48, 16)]
        return (a0, a1, a2, a3)

    def body(j, a):
        r0 = 4 * j
        for k in range(4):
            a = add_row(a, r0 + k)
        return a

    acc = lax.fori_loop(0, n_rows // 4, body, acc, unroll=False)
    for r in range(n_rows - n_rows % 4, n_rows):
        acc = add_row(acc, r)
    return acc


def _store_acc(acc_ref, i, acc):
    a0, a1, a2, a3 = acc
    acc_ref[i, pl.ds(0, 16)] = a0
    acc_ref[i, pl.ds(16, 16)] = a1
    acc_ref[i, pl.ds(32, 16)] = a2
    acc_ref[i, pl.ds(48, 16)] = a3


def _sc_pool_kernel(didx_hbm, qidx_hbm, table_hbm, d_out_hbm, q_out_hbm,
                    didx_v, qidx_v, rows0, rows1, rows2, rows3,
                    dacc_v, qacc_v, sem0, sem1, sem2, sem3):
    wid = lax.axis_index("s") * _NC + lax.axis_index("c")
    rows = (rows0, rows1, rows2, rows3)
    sems = (sem0, sem1, sem2, sem3)

    pltpu.sync_copy(didx_hbm.at[pl.ds(wid * _NDC, _NDC)], didx_v)
    pltpu.sync_copy(qidx_hbm.at[pl.ds(wid * _BPW, _BPW)], qidx_v)

    zero = jnp.zeros((16,), jnp.float32)
    z4 = (zero, zero, zero, zero)

    def d_start(chunk, b):
        return pltpu.async_copy(table_hbm.at[didx_v.at[chunk]], rows[b],
                                sems[b])

    def d_wait(chunk, b):
        pltpu.make_async_copy(table_hbm.at[didx_v.at[chunk]], rows[b],
                              sems[b]).wait()

    for b in range(_NBUF):
        d_start(b, b)

    def d_round(k, carry):
        acc = z4
        for b in range(_NBUF):
            chunk = _NBUF * k + b
            d_wait(chunk, b)
            acc = _pool_chunk(rows[b], _DCH, acc)
            if b % _DCHUNKS == _DCHUNKS - 1:
                _store_acc(dacc_v, 2 * k + b // _DCHUNKS, acc)
                acc = z4

            @pl.when(k < _NDC // _NBUF - 1)
            def _():
                d_start(chunk + _NBUF, b)

        return carry

    lax.fori_loop(0, _NDC // _NBUF, d_round, 0, unroll=False)
    pltpu.sync_copy(dacc_v, d_out_hbm.at[pl.ds(wid * _BPW, _BPW)])

    def q_start(i, b):
        return pltpu.async_copy(table_hbm.at[qidx_v.at[i]],
                                rows[b].at[pl.ds(0, _QUERY_LEN)], sems[b])

    def q_wait(i, b):
        pltpu.make_async_copy(table_hbm.at[qidx_v.at[i]],
                              rows[b].at[pl.ds(0, _QUERY_LEN)],
                              sems[b]).wait()

    for b in range(_NBUF):
        q_start(b, b)

    def q_round(k, carry):
        for b in range(_NBUF):
            i = _NBUF * k + b
            q_wait(i, b)
            acc = _pool_chunk(rows[b], _QUERY_LEN, z4)
            _store_acc(qacc_v, i, acc)

            @pl.when(k < _BPW // _NBUF - 1)
            def _():
                q_start(i + _NBUF, b)

        return carry

    lax.fori_loop(0, _BPW // _NBUF, q_round, 0, unroll=False)
    pltpu.sync_copy(qacc_v, q_out_hbm.at[pl.ds(wid * _BPW, _BPW)])


def _sc_pool(didx, qidx, table2):
    mesh = plsc.VectorSubcoreMesh(core_axis_name="c", subcore_axis_name="s")
    fn = functools.partial(
        pl.kernel,
        mesh=mesh,
        out_type=[
            jax.ShapeDtypeStruct((_B, _D), jnp.float32),
            jax.ShapeDtypeStruct((_B, _D), jnp.float32),
        ],
        scratch_types=[
            pltpu.VMEM((_NDC, _DCH), jnp.int32),
            pltpu.VMEM((_BPW, _QUERY_LEN), jnp.int32),
            pltpu.VMEM((_DCH, _P), jnp.float32),
            pltpu.VMEM((_DCH, _P), jnp.float32),
            pltpu.VMEM((_DCH, _P), jnp.float32),
            pltpu.VMEM((_DCH, _P), jnp.float32),
            pltpu.VMEM((_BPW, _D), jnp.float32),
            pltpu.VMEM((_BPW, _D), jnp.float32),
            pltpu.SemaphoreType.DMA,
            pltpu.SemaphoreType.DMA,
            pltpu.SemaphoreType.DMA,
            pltpu.SemaphoreType.DMA,
        ],
    )(_sc_pool_kernel)
    return fn(didx, qidx, table2)


def _tc_head_kernel(d_ref, q_ref, dw1_ref, db1_ref, dw2_ref, db2_ref,
                    qw1_ref, qb1_ref, qw2_ref, qb2_ref, out_ref):
    def dot_t(a, w):
        return lax.dot_general(a, w, (((1,), (1,)), ((), ())),
                               preferred_element_type=jnp.float32)

    d = d_ref[...] * (1.0 / _DOC_LEN)
    q = q_ref[...] * (1.0 / _QUERY_LEN)
    dh = jnp.maximum(dot_t(d, dw1_ref[...]) + db1_ref[...], 0.0)
    dp = dot_t(dh, dw2_ref[...]) + db2_ref[...]
    qh = jnp.maximum(dot_t(q, qw1_ref[...]) + qb1_ref[...], 0.0)
    qp = dot_t(qh, qw2_ref[...]) + qb2_ref[...]
    dn = jnp.maximum(jnp.sqrt(jnp.sum(dp * dp, axis=1, keepdims=True)), 1e-8)
    qn = jnp.maximum(jnp.sqrt(jnp.sum(qp * qp, axis=1, keepdims=True)), 1e-8)
    out_ref[...] = jnp.sum(dp * qp, axis=1, keepdims=True) / (dn * qn)


def _tc_head(d_sum, q_sum, d_w1, d_b1, d_w2, d_b2, q_w1, q_b1, q_w2, q_b2):
    return pl.pallas_call(
        _tc_head_kernel,
        out_shape=jax.ShapeDtypeStruct((_B, 1), jnp.float32),
    )(d_sum, q_sum, d_w1, d_b1.reshape(1, _P), d_w2, d_b2.reshape(1, _P),
      q_w1, q_b1.reshape(1, _D), q_w2, q_b2.reshape(1, _P))


def kernel(doc_ids, query_ids, table, d_w1, d_b1, d_w2, d_b2,
           q_w1, q_b1, q_w2, q_b2):
    doc_ids = doc_ids.astype(jnp.int32)
    query_ids = query_ids.astype(jnp.int32)
    table2 = _sc_relayout(table.T, table[_NVB * _VB:])
    didx = doc_ids.reshape(_B * _DCHUNKS, _DCH)
    d_sum, q_sum = _sc_pool(didx, query_ids, table2)
    sim = _tc_head(d_sum, q_sum, d_w1, d_b1, d_w2, d_b2,
                   q_w1, q_b1, q_w2, q_b2)
    return sim.reshape(_B)
